# Initial kernel scaffold; baseline (speedup 1.0000x reference)
import jax
import jax.numpy as jnp
from jax.experimental import pallas as pl


def kernel(x, edge_attr, edge_index, Wm0, bm0, Wa0, ba0, We0, be0, Wm1, bm1, Wa1, ba1, We1, be1, Wm2, bm2, Wa2, ba2, We2, be2, Wout, bout):
    raise NotImplementedError("write your pallas kernel here")


# speedup vs baseline: 2.4995x; 2.4995x over previous
import functools

import jax
import jax.numpy as jnp
from jax import lax
from jax.experimental import pallas as pl
from jax.experimental.pallas import tpu as pltpu
from jax.experimental.pallas import tpu_sc as plsc

N = 10000
E = 320000
D = 128
ED = 16

NC = 2
NS = 16
NW = NC * NS
EPT = E // NW
K = 80
NCHUNK = EPT // K
N_PAD = 10240
NPS = N_PAD // NS
NPC = 128
CPS = N_PAD // NS

_f32 = jnp.float32
_mesh = plsc.VectorSubcoreMesh(core_axis_name="c", subcore_axis_name="s")


def _zero_buf_2d(buf, rows):
    @pl.loop(0, rows)
    def _z(r):
        for j in range(D // 16):
            buf[r, pl.ds(j * 16, 16)] = jnp.zeros((16,), _f32)


@functools.partial(
    pl.kernel,
    out_type=(
        jax.ShapeDtypeStruct((NC, N_PAD, D), _f32),
        jax.ShapeDtypeStruct((NC, N_PAD), _f32),
    ),
    mesh=_mesh,
    scratch_types=[
        pltpu.VMEM_SHARED((N_PAD, D), _f32),
        pltpu.VMEM_SHARED((N_PAD,), _f32),
        pltpu.VMEM((K,), jnp.int32),
        pltpu.VMEM((K,), jnp.int32),
        pltpu.VMEM((K,), _f32),
        pltpu.VMEM((D,), _f32),
        pltpu.VMEM((K, D), _f32),
        pltpu.VMEM((K, D), _f32),
        pltpu.VMEM((K,), _f32),
        pltpu.VMEM((NPC, D), _f32),
        pltpu.VMEM((CPS,), _f32),
        pltpu.SemaphoreType.DMA,
        pltpu.SemaphoreType.DMA,
        pltpu.SemaphoreType.DMA,
    ],
)
def _sca0(p_hbm, ea_hbm, wme_hbm, src_hbm, dst_hbm, agg_out, cnt_out,
          agg_sh, cnt_sh, src_b, dst_b, ea_b, w_b, rows_b, m_b, ones_b,
          zero_b, zc_b, sem0, sem1, sem2):
    core = lax.axis_index("c")
    sid = lax.axis_index("s")
    wid = sid * NC + core

    _zero_buf_2d(zero_b, NPC)
    for j in range(CPS // 16):
        zc_b[pl.ds(j * 16, 16)] = jnp.zeros((16,), _f32)
    for j in range(K // 16):
        ones_b[pl.ds(j * 16, 16)] = jnp.full((16,), 1.0, _f32)
    pltpu.sync_copy(wme_hbm, w_b)
    for t in range(NPS // NPC):
        pltpu.sync_copy(zero_b, agg_sh.at[pl.ds(sid * NPS + t * NPC, NPC), :])
    pltpu.sync_copy(zc_b, cnt_sh.at[pl.ds(sid * CPS, CPS)])
    plsc.subcore_barrier()

    base0 = wid * EPT

    @pl.loop(0, NCHUNK)
    def _edges(i):
        base = base0 + i * K
        cp_s = pltpu.async_copy(src_hbm.at[pl.ds(base, K)], src_b, sem0)
        cp_d = pltpu.async_copy(dst_hbm.at[pl.ds(base, K)], dst_b, sem1)
        cp_e = pltpu.async_copy(ea_hbm.at[pl.ds(base, K)], ea_b, sem2)
        cp_s.wait()
        cp_g = pltpu.async_copy(p_hbm.at[src_b], rows_b, sem0)
        cp_g.wait()
        cp_e.wait()

        @pl.loop(0, K // 16)
        def _grp(g):
            v = ea_b[pl.ds(g * 16, 16)]
            for lane in range(16):
                s = v[lane]
                r = g * 16 + lane
                for j in range(D // 16):
                    sl = pl.ds(j * 16, 16)
                    m_b[r, sl] = jnp.maximum(rows_b[r, sl] + s * w_b[sl], 0.0)

        cp_d.wait()
        pltpu.sync_copy(m_b, agg_sh.at[dst_b], add=True)
        pltpu.sync_copy(ones_b, cnt_sh.at[dst_b], add=True)

    plsc.subcore_barrier()
    for t in range(NPS // NPC):
        r0 = sid * NPS + t * NPC
        pltpu.sync_copy(agg_sh.at[pl.ds(r0, NPC), :], zero_b)
        pltpu.sync_copy(zero_b, agg_out.at[core, pl.ds(r0, NPC), :])
    pltpu.sync_copy(cnt_sh.at[pl.ds(sid * CPS, CPS)], zc_b)
    pltpu.sync_copy(zc_b, cnt_out.at[core, pl.ds(sid * CPS, CPS)])


@functools.partial(
    pl.kernel,
    out_type=jax.ShapeDtypeStruct((NC, N_PAD, D), _f32),
    mesh=_mesh,
    scratch_types=[
        pltpu.VMEM_SHARED((N_PAD, D), _f32),
        pltpu.VMEM((K,), jnp.int32),
        pltpu.VMEM((K,), jnp.int32),
        pltpu.VMEM((K, D), _f32),
        pltpu.VMEM((K, D), _f32),
        pltpu.VMEM((K, D), _f32),
        pltpu.VMEM((NPC, D), _f32),
        pltpu.SemaphoreType.DMA,
        pltpu.SemaphoreType.DMA,
        pltpu.SemaphoreType.DMA,
    ],
)
def _sca(p_hbm, c_hbm, src_hbm, dst_hbm, agg_out,
         agg_sh, src_b, dst_b, rows_b, c_b, m_b, zero_b, sem0, sem1, sem2):
    core = lax.axis_index("c")
    sid = lax.axis_index("s")
    wid = sid * NC + core

    _zero_buf_2d(zero_b, NPC)
    for t in range(NPS // NPC):
        pltpu.sync_copy(zero_b, agg_sh.at[pl.ds(sid * NPS + t * NPC, NPC), :])
    plsc.subcore_barrier()

    base0 = wid * EPT

    @pl.loop(0, NCHUNK)
    def _edges(i):
        base = base0 + i * K
        cp_s = pltpu.async_copy(src_hbm.at[pl.ds(base, K)], src_b, sem0)
        cp_d = pltpu.async_copy(dst_hbm.at[pl.ds(base, K)], dst_b, sem1)
        cp_c = pltpu.async_copy(c_hbm.at[pl.ds(base, K), :], c_b, sem2)
        cp_s.wait()
        cp_g = pltpu.async_copy(p_hbm.at[src_b], rows_b, sem0)
        cp_g.wait()
        cp_c.wait()

        @pl.loop(0, K)
        def _row(r):
            for j in range(D // 16):
                sl = pl.ds(j * 16, 16)
                m_b[r, sl] = jnp.maximum(rows_b[r, sl] + c_b[r, sl], 0.0)

        cp_d.wait()
        pltpu.sync_copy(m_b, agg_sh.at[dst_b], add=True)

    plsc.subcore_barrier()
    for t in range(NPS // NPC):
        r0 = sid * NPS + t * NPC
        pltpu.sync_copy(agg_sh.at[pl.ds(r0, NPC), :], zero_b)
        pltpu.sync_copy(zero_b, agg_out.at[core, pl.ds(r0, NPC), :])


@functools.partial(
    pl.kernel,
    out_type=jax.ShapeDtypeStruct((E, ED), _f32),
    mesh=_mesh,
    scratch_types=[
        pltpu.VMEM((K,), jnp.int32),
        pltpu.VMEM((K,), jnp.int32),
        pltpu.VMEM((K,), _f32),
        pltpu.VMEM((ED,), _f32),
        pltpu.VMEM((K, D), _f32),
        pltpu.VMEM((K, D), _f32),
        pltpu.VMEM((K, ED), _f32),
        pltpu.SemaphoreType.DMA,
        pltpu.SemaphoreType.DMA,
        pltpu.SemaphoreType.DMA,
    ],
)
def _scb0(qsd_hbm, ea_hbm, wee_hbm, src_hbm, dst_hbm, ea_out,
          src_b, dst_b, ea_b, w_b, qs_b, qd_b, r_b, sem0, sem1, sem2):
    core = lax.axis_index("c")
    sid = lax.axis_index("s")
    wid = sid * NC + core
    pltpu.sync_copy(wee_hbm, w_b)
    base0 = wid * EPT

    @pl.loop(0, NCHUNK)
    def _edges(i):
        base = base0 + i * K
        cp_s = pltpu.async_copy(src_hbm.at[pl.ds(base, K)], src_b, sem0)
        cp_d = pltpu.async_copy(dst_hbm.at[pl.ds(base, K)], dst_b, sem1)
        cp_e = pltpu.async_copy(ea_hbm.at[pl.ds(base, K)], ea_b, sem2)
        cp_s.wait()
        cp_qs = pltpu.async_copy(qsd_hbm.at[src_b], qs_b, sem0)
        cp_d.wait()
        cp_qd = pltpu.async_copy(qsd_hbm.at[dst_b], qd_b, sem1)
        cp_qs.wait()
        cp_qd.wait()
        cp_e.wait()

        @pl.loop(0, K // 16)
        def _grp(g):
            v = ea_b[pl.ds(g * 16, 16)]
            for lane in range(16):
                r = g * 16 + lane
                r_b[r, :] = jnp.maximum(
                    qs_b[r, pl.ds(0, ED)] + qd_b[r, pl.ds(ED, ED)]
                    + v[lane] * w_b[:], 0.0)

        pltpu.sync_copy(r_b, ea_out.at[pl.ds(base, K), :])


@functools.partial(
    pl.kernel,
    out_type=jax.ShapeDtypeStruct((E, ED), _f32),
    mesh=_mesh,
    scratch_types=[
        pltpu.VMEM((K,), jnp.int32),
        pltpu.VMEM((K,), jnp.int32),
        pltpu.VMEM((K, D), _f32),
        pltpu.VMEM((K, D), _f32),
        pltpu.VMEM((K, ED), _f32),
        pltpu.SemaphoreType.DMA,
        pltpu.SemaphoreType.DMA,
        pltpu.SemaphoreType.DMA,
    ],
)
def _scb(qsd_hbm, r_hbm, src_hbm, dst_hbm, ea_out,
         src_b, dst_b, qs_b, qd_b, r_b, sem0, sem1, sem2):
    core = lax.axis_index("c")
    sid = lax.axis_index("s")
    wid = sid * NC + core
    base0 = wid * EPT

    @pl.loop(0, NCHUNK)
    def _edges(i):
        base = base0 + i * K
        cp_s = pltpu.async_copy(src_hbm.at[pl.ds(base, K)], src_b, sem0)
        cp_d = pltpu.async_copy(dst_hbm.at[pl.ds(base, K)], dst_b, sem1)
        cp_r = pltpu.async_copy(r_hbm.at[pl.ds(base, K), :], r_b, sem2)
        cp_s.wait()
        cp_qs = pltpu.async_copy(qsd_hbm.at[src_b], qs_b, sem0)
        cp_d.wait()
        cp_qd = pltpu.async_copy(qsd_hbm.at[dst_b], qd_b, sem1)
        cp_qs.wait()
        cp_qd.wait()
        cp_r.wait()

        @pl.loop(0, K)
        def _row(r):
            r_b[r, :] = jnp.maximum(
                qs_b[r, pl.ds(0, ED)] + qd_b[r, pl.ds(ED, ED)] + r_b[r, :], 0.0)

        pltpu.sync_copy(r_b, ea_out.at[pl.ds(base, K), :])


NB = 2000
BE = 3200


def _dot(a, b):
    return jnp.dot(a, b, preferred_element_type=_f32)


def _p_body(x_ref, w_ref, b_ref, o_ref):
    o_ref[...] = _dot(x_ref[...], w_ref[...]) + b_ref[...]


_p_call = pl.pallas_call(
    _p_body,
    grid=(N // NB,),
    in_specs=[
        pl.BlockSpec((NB, D), lambda i: (i, 0)),
        pl.BlockSpec((D, D), lambda i: (0, 0)),
        pl.BlockSpec((1, D), lambda i: (0, 0)),
    ],
    out_specs=pl.BlockSpec((NB, D), lambda i: (i, 0)),
    out_shape=jax.ShapeDtypeStruct((N, D), _f32),
)


def _node_body(a0, a1, c0, c1, x, waa, wax, ba, wmx, bm, wesd, besd,
               xo, po, qsdo):
    inv = 1.0 / jnp.maximum(c0[...] + c1[...], 1.0)
    agg = (a0[0] + a1[0]) * inv
    xn = jnp.maximum(_dot(agg, waa[...]) + _dot(x[...], wax[...]) + ba[...], 0.0)
    xo[...] = xn
    po[...] = _dot(xn, wmx[...]) + bm[...]
    qsdo[...] = _dot(xn, wesd[...]) + besd[...]


_node_call = pl.pallas_call(
    _node_body,
    grid=(N // NB,),
    in_specs=[
        pl.BlockSpec((1, NB, D), lambda i: (0, i, 0)),
        pl.BlockSpec((1, NB, D), lambda i: (1, i, 0)),
        pl.BlockSpec((NB, 1), lambda i: (i, 0)),
        pl.BlockSpec((NB, 1), lambda i: (i, 0)),
        pl.BlockSpec((NB, D), lambda i: (i, 0)),
        pl.BlockSpec((D, D), lambda i: (0, 0)),
        pl.BlockSpec((D, D), lambda i: (0, 0)),
        pl.BlockSpec((1, D), lambda i: (0, 0)),
        pl.BlockSpec((D, D), lambda i: (0, 0)),
        pl.BlockSpec((1, D), lambda i: (0, 0)),
        pl.BlockSpec((D, D), lambda i: (0, 0)),
        pl.BlockSpec((1, D), lambda i: (0, 0)),
    ],
    out_specs=[
        pl.BlockSpec((NB, D), lambda i: (i, 0)),
        pl.BlockSpec((NB, D), lambda i: (i, 0)),
        pl.BlockSpec((NB, D), lambda i: (i, 0)),
    ],
    out_shape=[
        jax.ShapeDtypeStruct((N, D), _f32),
        jax.ShapeDtypeStruct((N, D), _f32),
        jax.ShapeDtypeStruct((N, D), _f32),
    ],
)


def _final_body(a0, a1, c0, c1, x, waa, wax, ba, wo, bo, o):
    inv = 1.0 / jnp.maximum(c0[...] + c1[...], 1.0)
    agg = (a0[0] + a1[0]) * inv
    xn = jnp.maximum(_dot(agg, waa[...]) + _dot(x[...], wax[...]) + ba[...], 0.0)
    o[...] = jnp.maximum(_dot(xn, wo[...]) + bo[...], 0.0)


_final_call = pl.pallas_call(
    _final_body,
    grid=(N // NB,),
    in_specs=[
        pl.BlockSpec((1, NB, D), lambda i: (0, i, 0)),
        pl.BlockSpec((1, NB, D), lambda i: (1, i, 0)),
        pl.BlockSpec((NB, 1), lambda i: (i, 0)),
        pl.BlockSpec((NB, 1), lambda i: (i, 0)),
        pl.BlockSpec((NB, D), lambda i: (i, 0)),
        pl.BlockSpec((D, D), lambda i: (0, 0)),
        pl.BlockSpec((D, D), lambda i: (0, 0)),
        pl.BlockSpec((1, D), lambda i: (0, 0)),
        pl.BlockSpec((D, D), lambda i: (0, 0)),
        pl.BlockSpec((1, D), lambda i: (0, 0)),
    ],
    out_specs=pl.BlockSpec((NB, D), lambda i: (i, 0)),
    out_shape=jax.ShapeDtypeStruct((N, D), _f32),
)


def _ed_body(ea, wme, wee, co, ro):
    e = ea[...]
    co[...] = _dot(e, wme[...])
    ro[...] = _dot(e, wee[...])


_ed_call = pl.pallas_call(
    _ed_body,
    grid=(E // BE,),
    in_specs=[
        pl.BlockSpec((BE, ED), lambda i: (i, 0)),
        pl.BlockSpec((ED, D), lambda i: (0, 0)),
        pl.BlockSpec((ED, ED), lambda i: (0, 0)),
    ],
    out_specs=[
        pl.BlockSpec((BE, D), lambda i: (i, 0)),
        pl.BlockSpec((BE, ED), lambda i: (i, 0)),
    ],
    out_shape=[
        jax.ShapeDtypeStruct((E, D), _f32),
        jax.ShapeDtypeStruct((E, ED), _f32),
    ],
)


def kernel(x, edge_attr, edge_index, Wm0, bm0, Wa0, ba0, We0, be0,
           Wm1, bm1, Wa1, ba1, We1, be1, Wm2, bm2, Wa2, ba2, We2, be2,
           Wout, bout):
    src = edge_index[0]
    dst = edge_index[1]

    z96 = jnp.zeros((D, D - 2 * ED), _f32)
    zb = jnp.zeros((1, D - ED), _f32)
    WeSD0 = jnp.concatenate([We0[:D], We0[D:2 * D], z96], axis=1)
    beSD0 = jnp.concatenate([be0.reshape(1, ED), zb], axis=1)
    WeSD1 = jnp.concatenate([We1[:D], We1[D:2 * D], z96], axis=1)
    beSD1 = jnp.concatenate([be1.reshape(1, ED), zb], axis=1)
    P0 = _p_call(x, Wm0[:D], bm0.reshape(1, D))
    aggp, cntp = _sca0(P0, edge_attr, Wm0[D], src, dst)
    c0 = cntp[0, :N].reshape(N, 1)
    c1 = cntp[1, :N].reshape(N, 1)
    x1, P1, Qsd0 = _node_call(
        aggp, aggp, c0, c1, x,
        Wa0[:D], Wa0[D:], ba0.reshape(1, D),
        Wm1[:D], bm1.reshape(1, D),
        WeSD0, beSD0)
    ea1 = _scb0(Qsd0, edge_attr, We0[2 * D], src, dst)

    C1, R1 = _ed_call(ea1, Wm1[D:], We1[2 * D:])
    aggp1 = _sca(P1, C1, src, dst)
    x2, P2, Qsd1 = _node_call(
        aggp1, aggp1, c0, c1, x1,
        Wa1[:D], Wa1[D:], ba1.reshape(1, D),
        Wm2[:D], bm2.reshape(1, D),
        WeSD1, beSD1)
    ea2 = _scb(Qsd1, R1, src, dst)

    C2, _unused = _ed_call(ea2, Wm2[D:], We2[2 * D:])
    aggp2 = _sca(P2, C2, src, dst)
    out = _final_call(
        aggp2, aggp2, c0, c1, x2,
        Wa2[:D], Wa2[D:], ba2.reshape(1, D),
        Wout, bout.reshape(1, D))
    return out
